# Initial kernel scaffold; baseline (speedup 1.0000x reference)
import jax
import jax.numpy as jnp
from jax.experimental import pallas as pl


def kernel(input_ids, adapt_emb0, adapt_emb1, adapt_proj1, fc_f1, fc_f2, fc_basis, fc_hash, fc_W, fc_b, rh_tables, hash_a, hash_b, rh_W, rh_b):
    raise NotImplementedError("write your pallas kernel here")


# speedup vs baseline: 12.8749x; 12.8749x over previous
import functools

import jax
import jax.numpy as jnp
from jax import lax
from jax.experimental import pallas as pl
from jax.experimental.pallas import tpu as pltpu
from jax.experimental.pallas import tpu_sc as plsc

_VOCAB = 1000000
_EMBED = 64
_CUT1 = 100000
_CUT2 = 500000
_ADAPT_CUT = 20000
_MID_VOCAB = _CUT2 - _CUT1
_F1_SIZE = 633
_F2_SIZE = 632
_NFC = 256
_NHB = 100000
_NHF = 4

_OFF_F1 = _CUT1
_OFF_F2 = _OFF_F1 + 640
_OFF_B = _OFF_F2 + 640
_ZROW = _OFF_B + 256
_OFF_T = _ZROW + 32
_NROWS = _OFF_T + _NHF * _NHB

_CHUNK = 256
_IDXB = 128


def _mm_block(x, w):
    n, k = x.shape
    bn = 8000

    def body(x_ref, w_ref, o_ref):
        o_ref[...] = jnp.dot(x_ref[...], w_ref[...],
                             preferred_element_type=jnp.float32)

    return pl.pallas_call(
        body,
        grid=(n // bn,),
        in_specs=[pl.BlockSpec((bn, k), lambda i: (i, jnp.int32(0))),
                  pl.BlockSpec((k, _EMBED), lambda i: (jnp.int32(0), jnp.int32(0)))],
        out_specs=pl.BlockSpec((bn, _EMBED), lambda i: (i, jnp.int32(0))),
        out_shape=jax.ShapeDtypeStruct((n, _EMBED), jnp.float32),
    )(x, w)


def _mm_small(x, w, bias):
    n = x.shape[0]

    def body(x_ref, w_ref, b_ref, o_ref):
        o_ref[...] = jnp.dot(x_ref[...], w_ref[...],
                             preferred_element_type=jnp.float32) + b_ref[...]

    return pl.pallas_call(
        body,
        out_shape=jax.ShapeDtypeStruct((n, _EMBED), jnp.float32),
    )(x, w, bias)


def _mm_rh(x, w4, b4):
    bn = 20000

    def body(x_ref, w_ref, b_ref, o_ref):
        o_ref[0] = jnp.dot(x_ref[0], w_ref[0],
                           preferred_element_type=jnp.float32) + b_ref[0]

    return pl.pallas_call(
        body,
        grid=(_NHF, _NHB // bn),
        in_specs=[pl.BlockSpec((1, bn, 16), lambda i, j: (i, j, jnp.int32(0))),
                  pl.BlockSpec((1, 16, _EMBED), lambda i, j: (i, jnp.int32(0), jnp.int32(0))),
                  pl.BlockSpec((1, 1, _EMBED), lambda i, j: (i, jnp.int32(0), jnp.int32(0)))],
        out_specs=pl.BlockSpec((1, bn, _EMBED), lambda i, j: (i, j, jnp.int32(0))),
        out_shape=jax.ShapeDtypeStruct((_NHF, _NHB, _EMBED), jnp.float32),
    )(x, w4, b4)


def _sc_lookup(table, ids32, fch32, params, n_tokens):
    info = plsc.get_sparse_core_info()
    nc, ns = info.num_cores, info.num_subcores
    nw = nc * ns
    per_w = n_tokens // nw
    n_chunks = per_w // _CHUNK
    mesh = plsc.VectorSubcoreMesh(core_axis_name="c", subcore_axis_name="s")

    @functools.partial(
        pl.kernel,
        out_type=jax.ShapeDtypeStruct((n_tokens, _EMBED), jnp.float32),
        mesh=mesh,
        scratch_types=[
            pltpu.VMEM((_CHUNK,), jnp.int32),
            pltpu.VMEM((12 * 16,), jnp.int32),
            pltpu.VMEM((_CHUNK,), jnp.int32),
            pltpu.VMEM((_CHUNK,), jnp.int32),
            pltpu.VMEM((4 * _CHUNK,), jnp.int32),
            pltpu.VMEM((4 * _CHUNK, _EMBED), jnp.float32),
            pltpu.VMEM((_CHUNK, _EMBED), jnp.float32),
            pltpu.SemaphoreType.DMA,
        ],
        compiler_params=pltpu.CompilerParams(use_tc_tiling_on_sc=False),
    )
    def k(table_hbm, ids_hbm, fch_hbm, par_hbm, out_hbm,
          ids_v, par_v, hidx_v, hv_v, idx_v, rows_v, out_v, sem):
        wid = lax.axis_index("s") * jnp.int32(nc) + lax.axis_index("c")
        base = wid * jnp.int32(per_w)
        pltpu.sync_copy(par_hbm, par_v)

        c633 = jnp.full((16,), _F1_SIZE, jnp.int32)
        cnhb = jnp.full((16,), _NHB, jnp.int32)

        def chunk(ci, carry):
            off = base + ci * _CHUNK
            pltpu.sync_copy(ids_hbm.at[pl.ds(off, _CHUNK)], ids_v)

            def pass1(v, c):
                ids = ids_v[pl.ds(v * 16, 16)]
                t2 = jnp.logical_and(ids >= _CUT1, ids < _CUT2)
                hidx_v[pl.ds(v * 16, 16)] = jnp.where(t2, ids - _CUT1, 0)
                return c
            lax.fori_loop(jnp.int32(0), jnp.int32(_CHUNK // 16), pass1, jnp.int32(0))

            cps = [pltpu.async_copy(
                       fch_hbm.at[hidx_v.at[pl.ds(j * _IDXB, _IDXB)]],
                       hv_v.at[pl.ds(j * _IDXB, _IDXB)], sem)
                   for j in range(_CHUNK // _IDXB)]
            for cp in cps:
                cp.wait()

            def pass2(v, c):
                o16 = v * 16
                ids = ids_v[pl.ds(o16, 16)]
                hv = hv_v[pl.ds(o16, 16)]
                t1 = ids < _CUT1
                t3 = ids >= _CUT2
                t2 = jnp.logical_and(ids >= _CUT1, ids < _CUT2)
                idf = ids - _CUT1
                q = lax.div(idf, c633)
                r = idf - q * c633
                idr = ids - _CUT2
                hi = idr >> 8
                lo = idr & 255
                hs = []
                for i in range(_NHF):
                    a256 = par_v[pl.ds(i * 16, 16)]
                    a1 = par_v[pl.ds((4 + i) * 16, 16)]
                    bi = par_v[pl.ds((8 + i) * 16, 16)]
                    hs.append(lax.rem(hi * a256 + lo * a1 + bi, cnhb))
                r0 = jnp.where(t1, ids,
                               jnp.where(t2, _OFF_F1 + r, _OFF_T + hs[0]))
                r1 = jnp.where(t2, _OFF_F2 + q,
                               jnp.where(t3, (_OFF_T + _NHB) + hs[1], _ZROW))
                r2 = jnp.where(t2, hv,
                               jnp.where(t3, (_OFF_T + 2 * _NHB) + hs[2],
                                         _ZROW))
                r3 = jnp.where(t3, (_OFF_T + 3 * _NHB) + hs[3], _ZROW)
                idx_v[pl.ds(o16, 16)] = r0
                idx_v[pl.ds(_CHUNK + o16, 16)] = r1
                idx_v[pl.ds(2 * _CHUNK + o16, 16)] = r2
                idx_v[pl.ds(3 * _CHUNK + o16, 16)] = r3
                return c
            lax.fori_loop(jnp.int32(0), jnp.int32(_CHUNK // 16), pass2, jnp.int32(0))

            cps = [pltpu.async_copy(
                       table_hbm.at[idx_v.at[pl.ds(j * _IDXB, _IDXB)]],
                       rows_v.at[pl.ds(j * _IDXB, _IDXB)], sem)
                   for j in range(4 * _CHUNK // _IDXB)]
            for cp in cps:
                cp.wait()

            def pass3(t, c):
                for col in range(_EMBED // 16):
                    sl = pl.ds(col * 16, 16)
                    out_v[t, sl] = (rows_v[t, sl]
                                    + rows_v[_CHUNK + t, sl]
                                    + rows_v[2 * _CHUNK + t, sl]
                                    + rows_v[3 * _CHUNK + t, sl])
                return c
            lax.fori_loop(jnp.int32(0), jnp.int32(_CHUNK), pass3, jnp.int32(0))

            pltpu.sync_copy(out_v, out_hbm.at[pl.ds(off, _CHUNK)])
            return carry
        lax.fori_loop(jnp.int32(0), jnp.int32(n_chunks), chunk, jnp.int32(0))

    return k(table, ids32, fch32, params)


def kernel(input_ids, adapt_emb0, adapt_emb1, adapt_proj1, fc_f1, fc_f2,
           fc_basis, fc_hash, fc_W, fc_b, rh_tables, hash_a, hash_b,
           rh_W, rh_b):
    orig_shape = input_ids.shape
    n_tokens = orig_shape[0] * orig_shape[1]
    f32 = jnp.float32

    e1 = _mm_block(adapt_emb1.astype(f32), adapt_proj1.astype(f32))

    xs = jnp.concatenate([
        jnp.pad(fc_f1.astype(f32), ((0, 7), (0, 32))),
        jnp.pad(fc_f2.astype(f32), ((0, 8), (32, 0))),
        fc_basis.astype(f32) * 0.1,
        jnp.zeros((32, _EMBED), f32),
    ], axis=0)
    bias_small = jnp.where((jnp.arange(1568) < 640)[:, None],
                           fc_b.astype(f32)[None, :], 0.0).astype(f32)
    small = _mm_small(xs, fc_W.astype(f32), bias_small)

    w4 = rh_W.astype(f32).reshape(_NHF, 16, _EMBED)
    b4 = jnp.concatenate([rh_b.astype(f32)[None, None, :],
                          jnp.zeros((_NHF - 1, 1, _EMBED), f32)], axis=0)
    t4 = _mm_rh(rh_tables.astype(f32), w4, b4)

    table = jnp.concatenate([
        adapt_emb0.astype(f32),
        e1,
        small,
        t4.reshape(_NHF * _NHB, _EMBED),
    ], axis=0)

    ids32 = input_ids.reshape(-1).astype(jnp.int32)
    fch32 = (fc_hash + _OFF_B).astype(jnp.int32)
    a64 = hash_a.astype(jnp.int64)
    b64 = hash_b.astype(jnp.int64)
    pvals = jnp.concatenate([(a64 * 256) % _NHB, a64 % _NHB, b64 % _NHB])
    params = jnp.broadcast_to(pvals.astype(jnp.int32)[:, None],
                              (12, 16)).reshape(12 * 16)

    out = _sc_lookup(table, ids32, fch32, params, n_tokens)
    return out.reshape(orig_shape[0], orig_shape[1], _EMBED).astype(jnp.float64)


# speedup vs baseline: 12.8800x; 1.0004x over previous
import functools

import jax
import jax.numpy as jnp
from jax import lax
from jax.experimental import pallas as pl
from jax.experimental.pallas import tpu as pltpu
from jax.experimental.pallas import tpu_sc as plsc

_VOCAB = 1000000
_EMBED = 64
_CUT1 = 100000
_CUT2 = 500000
_ADAPT_CUT = 20000
_MID_VOCAB = _CUT2 - _CUT1
_F1_SIZE = 633
_F2_SIZE = 632
_NFC = 256
_NHB = 100000
_NHF = 4

_OFF_F1 = _CUT1
_OFF_F2 = _OFF_F1 + 640
_OFF_B = _OFF_F2 + 640
_ZROW = _OFF_B + 256
_OFF_T = _ZROW + 32
_NROWS = _OFF_T + _NHF * _NHB

_CHUNK = 256
_IDXB = 128


def _mm_block(x, w):
    n, k = x.shape
    bn = 4000

    def body(x_ref, w_ref, o_ref):
        o_ref[...] = jnp.dot(x_ref[...], w_ref[...],
                             preferred_element_type=jnp.float32)

    return pl.pallas_call(
        body,
        grid=(n // bn,),
        in_specs=[pl.BlockSpec((bn, k), lambda i: (i, jnp.int32(0))),
                  pl.BlockSpec((k, _EMBED), lambda i: (jnp.int32(0), jnp.int32(0)))],
        out_specs=pl.BlockSpec((bn, _EMBED), lambda i: (i, jnp.int32(0))),
        out_shape=jax.ShapeDtypeStruct((n, _EMBED), jnp.float32),
    )(x, w)


def _mm_small(x, w, bias):
    n = x.shape[0]

    def body(x_ref, w_ref, b_ref, o_ref):
        o_ref[...] = jnp.dot(x_ref[...], w_ref[...],
                             preferred_element_type=jnp.float32) + b_ref[...]

    return pl.pallas_call(
        body,
        out_shape=jax.ShapeDtypeStruct((n, _EMBED), jnp.float32),
    )(x, w, bias)


def _mm_rh(x, w4, b4):
    bn = 10000

    def body(x_ref, w_ref, b_ref, o_ref):
        o_ref[0] = jnp.dot(x_ref[0], w_ref[0],
                           preferred_element_type=jnp.float32) + b_ref[0]

    return pl.pallas_call(
        body,
        grid=(_NHF, _NHB // bn),
        in_specs=[pl.BlockSpec((1, bn, 16), lambda i, j: (i, j, jnp.int32(0))),
                  pl.BlockSpec((1, 16, _EMBED), lambda i, j: (i, jnp.int32(0), jnp.int32(0))),
                  pl.BlockSpec((1, 1, _EMBED), lambda i, j: (i, jnp.int32(0), jnp.int32(0)))],
        out_specs=pl.BlockSpec((1, bn, _EMBED), lambda i, j: (i, j, jnp.int32(0))),
        out_shape=jax.ShapeDtypeStruct((_NHF, _NHB, _EMBED), jnp.float32),
    )(x, w4, b4)


def _sc_lookup(table, ids32, fch32, params, n_tokens):
    info = plsc.get_sparse_core_info()
    nc, ns = info.num_cores, info.num_subcores
    nw = nc * ns
    per_w = n_tokens // nw
    n_chunks = per_w // _CHUNK
    mesh = plsc.VectorSubcoreMesh(core_axis_name="c", subcore_axis_name="s")

    @functools.partial(
        pl.kernel,
        out_type=jax.ShapeDtypeStruct((n_tokens, _EMBED), jnp.float32),
        mesh=mesh,
        scratch_types=[
            pltpu.VMEM((_CHUNK,), jnp.int32),
            pltpu.VMEM((12 * 16,), jnp.int32),
            pltpu.VMEM((_CHUNK,), jnp.int32),
            pltpu.VMEM((_CHUNK,), jnp.int32),
            pltpu.VMEM((4 * _CHUNK,), jnp.int32),
            pltpu.VMEM((4 * _CHUNK, _EMBED), jnp.float32),
            pltpu.VMEM((_CHUNK, _EMBED), jnp.float32),
            pltpu.SemaphoreType.DMA,
        ],
        compiler_params=pltpu.CompilerParams(use_tc_tiling_on_sc=False),
    )
    def k(table_hbm, ids_hbm, fch_hbm, par_hbm, out_hbm,
          ids_v, par_v, hidx_v, hv_v, idx_v, rows_v, out_v, sem):
        wid = lax.axis_index("s") * jnp.int32(nc) + lax.axis_index("c")
        base = wid * jnp.int32(per_w)
        pltpu.sync_copy(par_hbm, par_v)

        def chunk(ci, carry):
            off = base + ci * _CHUNK
            pltpu.sync_copy(ids_hbm.at[pl.ds(off, _CHUNK)], ids_v)

            @plsc.parallel_loop(jnp.int32(0), jnp.int32(_CHUNK // 16), jnp.int32(1), unroll=4)
            def pass1(v):
                v16 = v * jnp.int32(16)
                ids = ids_v[pl.ds(v16, 16)]
                t2 = jnp.logical_and(ids >= _CUT1, ids < _CUT2)
                hidx_v[pl.ds(v16, 16)] = jnp.where(t2, ids - _CUT1, 0)

            cps = [pltpu.async_copy(
                       fch_hbm.at[hidx_v.at[pl.ds(j * _IDXB, _IDXB)]],
                       hv_v.at[pl.ds(j * _IDXB, _IDXB)], sem)
                   for j in range(_CHUNK // _IDXB)]
            for cp in cps:
                cp.wait()

            @plsc.parallel_loop(jnp.int32(0), jnp.int32(_CHUNK // 16), jnp.int32(1), unroll=2)
            def pass2(v):
                o16 = v * jnp.int32(16)
                ids = ids_v[pl.ds(o16, 16)]
                hv = hv_v[pl.ds(o16, 16)]
                t1 = ids < _CUT1
                t3 = ids >= _CUT2
                t2 = jnp.logical_and(ids >= _CUT1, ids < _CUT2)
                idf = ids - _CUT1
                q = (idf.astype(jnp.float32)
                     * jnp.float32(1.0 / _F1_SIZE)).astype(jnp.int32)
                r = idf - q * _F1_SIZE
                q = (q + jnp.where(r >= _F1_SIZE, 1, 0)
                     - jnp.where(r < 0, 1, 0))
                r = idf - q * _F1_SIZE
                idr = ids - _CUT2
                hi = idr >> 8
                lo = idr & 255
                hs = []
                for i in range(_NHF):
                    a256 = par_v[pl.ds(i * 16, 16)]
                    a1 = par_v[pl.ds((4 + i) * 16, 16)]
                    bi = par_v[pl.ds((8 + i) * 16, 16)]
                    x = hi * a256 + lo * a1 + bi
                    qh = (x.astype(jnp.float32)
                          * jnp.float32(1e-5)).astype(jnp.int32)
                    h = x - qh * _NHB
                    h = h + jnp.where(h < 0, _NHB, 0)
                    h = h - jnp.where(h >= _NHB, _NHB, 0)
                    hs.append(h)
                r0 = jnp.where(t1, ids,
                               jnp.where(t2, _OFF_F1 + r, _OFF_T + hs[0]))
                r1 = jnp.where(t2, _OFF_F2 + q,
                               jnp.where(t3, (_OFF_T + _NHB) + hs[1], _ZROW))
                r2 = jnp.where(t2, hv,
                               jnp.where(t3, (_OFF_T + 2 * _NHB) + hs[2],
                                         _ZROW))
                r3 = jnp.where(t3, (_OFF_T + 3 * _NHB) + hs[3], _ZROW)
                idx_v[pl.ds(o16, 16)] = r0
                idx_v[pl.ds(_CHUNK + o16, 16)] = r1
                idx_v[pl.ds(2 * _CHUNK + o16, 16)] = r2
                idx_v[pl.ds(3 * _CHUNK + o16, 16)] = r3

            cps = [pltpu.async_copy(
                       table_hbm.at[idx_v.at[pl.ds(j * _IDXB, _IDXB)]],
                       rows_v.at[pl.ds(j * _IDXB, _IDXB)], sem)
                   for j in range(4 * _CHUNK // _IDXB)]
            for cp in cps:
                cp.wait()

            @plsc.parallel_loop(jnp.int32(0), jnp.int32(_CHUNK), jnp.int32(1), unroll=4)
            def pass3(t):
                for col in range(_EMBED // 16):
                    sl = pl.ds(col * 16, 16)
                    out_v[t, sl] = (rows_v[t, sl]
                                    + rows_v[_CHUNK + t, sl]
                                    + rows_v[2 * _CHUNK + t, sl]
                                    + rows_v[3 * _CHUNK + t, sl])

            pltpu.sync_copy(out_v, out_hbm.at[pl.ds(off, _CHUNK)])
            return carry
        lax.fori_loop(jnp.int32(0), jnp.int32(n_chunks), chunk, jnp.int32(0))

    return k(table, ids32, fch32, params)


def kernel(input_ids, adapt_emb0, adapt_emb1, adapt_proj1, fc_f1, fc_f2,
           fc_basis, fc_hash, fc_W, fc_b, rh_tables, hash_a, hash_b,
           rh_W, rh_b):
    orig_shape = input_ids.shape
    n_tokens = orig_shape[0] * orig_shape[1]
    with jax.enable_x64(False):
        out = _kernel_32(input_ids, adapt_emb0, adapt_emb1, adapt_proj1,
                         fc_f1, fc_f2, fc_basis, fc_hash, fc_W, fc_b,
                         rh_tables, hash_a, hash_b, rh_W, rh_b, n_tokens)
    return out.reshape(orig_shape[0], orig_shape[1], _EMBED).astype(jnp.float64)


def _kernel_32(input_ids, adapt_emb0, adapt_emb1, adapt_proj1, fc_f1, fc_f2,
               fc_basis, fc_hash, fc_W, fc_b, rh_tables, hash_a, hash_b,
               rh_W, rh_b, n_tokens):
    f32 = jnp.float32

    e1 = _mm_block(adapt_emb1.astype(f32), adapt_proj1.astype(f32))

    xs = jnp.concatenate([
        jnp.pad(fc_f1.astype(f32), ((0, 7), (0, 32))),
        jnp.pad(fc_f2.astype(f32), ((0, 8), (32, 0))),
        fc_basis.astype(f32) * 0.1,
        jnp.zeros((32, _EMBED), f32),
    ], axis=0)
    bias_small = jnp.where((jnp.arange(1568) < 640)[:, None],
                           fc_b.astype(f32)[None, :], 0.0).astype(f32)
    small = _mm_small(xs, fc_W.astype(f32), bias_small)

    w4 = rh_W.astype(f32).reshape(_NHF, 16, _EMBED)
    b4 = jnp.concatenate([rh_b.astype(f32)[None, None, :],
                          jnp.zeros((_NHF - 1, 1, _EMBED), f32)], axis=0)
    t4 = _mm_rh(rh_tables.astype(f32), w4, b4)

    table = jnp.concatenate([
        adapt_emb0.astype(f32),
        e1,
        small,
        t4.reshape(_NHF * _NHB, _EMBED),
    ], axis=0)

    ids32 = input_ids.reshape(-1).astype(jnp.int32)
    fch32 = fc_hash.astype(jnp.int32) + _OFF_B
    a32 = hash_a.astype(jnp.int32)
    b32 = hash_b.astype(jnp.int32)
    pvals = jnp.concatenate([(a32 * 256) % _NHB, a32 % _NHB, b32 % _NHB])
    params = jnp.broadcast_to(pvals[:, None], (12, 16)).reshape(12 * 16)

    return _sc_lookup(table, ids32, fch32, params, n_tokens)
